# baseline (speedup 1.0000x reference)
import jax
import jax.numpy as jnp
from jax import lax
from jax.experimental import pallas as pl
from jax.experimental.pallas import tpu as pltpu
from jax.experimental.pallas import tpu_sc as plsc

_CONTEXT = 8192
_DIM = 64
_VOCAB = 1000000
_LANES = 16
_NUM_WORKERS = 32
_BPW = _CONTEXT // _NUM_WORKERS


def _emb_body(x_hbm, t_hbm, pos_hbm, out_hbm,
              xv, rows_v, posb, gsem, psem, xsem):
    wid = lax.axis_index("s") * 2 + lax.axis_index("c")
    base = wid * _BPW

    pltpu.async_copy(x_hbm.at[pl.ds(base, _BPW)], xv, xsem).wait()
    pos_cp = pltpu.async_copy(pos_hbm.at[pl.ds(base, _BPW), :], posb, psem)

    def fire_step(i, carry):
        j0 = i * _LANES
        v = xv[pl.ds(j0, _LANES)]
        tvec = lax.shift_right_logical(v, 3)
        svec = lax.bitwise_and(v, 7)
        for jo in range(_LANES):
            pltpu.async_copy(
                t_hbm.at[tvec[jo], svec[jo]], rows_v.at[j0 + jo], gsem)
        return carry

    lax.fori_loop(0, _BPW // _LANES, fire_step, 0)

    pltpu.make_async_copy(pos_hbm.at[pl.ds(0, _BPW), :], rows_v, gsem).wait()
    pos_cp.wait()

    def add_step(i, carry):
        for u in range(4):
            j = i * 4 + u
            for cc in range(_DIM // _LANES):
                sl = pl.ds(cc * _LANES, _LANES)
                rows_v[j, sl] = rows_v[j, sl] + posb[j, sl]
        return carry

    lax.fori_loop(0, _BPW // 4, add_step, 0)

    pltpu.sync_copy(rows_v, out_hbm.at[pl.ds(base, _BPW), :])


def kernel(x, table, pos_enc):
    table3 = table.reshape(_VOCAB // 8, 8, _DIM)
    mesh = plsc.VectorSubcoreMesh(core_axis_name="c", subcore_axis_name="s")
    f = pl.kernel(
        _emb_body,
        mesh=mesh,
        compiler_params=pltpu.CompilerParams(use_tc_tiling_on_sc=False),
        out_type=jax.ShapeDtypeStruct((_CONTEXT, _DIM), jnp.float32),
        scratch_types=[
            pltpu.VMEM((_BPW,), jnp.int32),
            pltpu.VMEM((_BPW, _DIM), jnp.float32),
            pltpu.VMEM((_BPW, _DIM), jnp.float32),
            pltpu.SemaphoreType.DMA,
            pltpu.SemaphoreType.DMA,
            pltpu.SemaphoreType.DMA,
        ],
    )
    return f(x, table3, pos_enc)


# speedup vs baseline: 2.3709x; 2.3709x over previous
import jax
import jax.numpy as jnp
from jax import lax
from jax.experimental import pallas as pl
from jax.experimental.pallas import tpu as pltpu
from jax.experimental.pallas import tpu_sc as plsc

_CONTEXT = 8192
_DIM = 64
_VOCAB = 1000000
_LANES = 16
_NUM_WORKERS = 32
_BPW = _CONTEXT // _NUM_WORKERS
_CHUNK = 16
_NCHUNKS = _BPW // _CHUNK


_NBUF = 3


def _emb_body(x_hbm, t_hbm, pos_hbm, out_hbm,
              xv, gb0, gb1, gb2, rows_v, posb, gsem0, gsem1, gsem2,
              psem, xsem):
    wid = lax.axis_index("s") * 2 + lax.axis_index("c")
    base = wid * _BPW

    t3 = t_hbm

    pltpu.async_copy(x_hbm.at[pl.ds(base, _BPW)], xv, xsem).wait()
    pos_cp = pltpu.async_copy(pos_hbm.at[pl.ds(base, _BPW), :], posb, psem)

    gbufs = (gb0, gb1, gb2)
    gsems = (gsem0, gsem1, gsem2)

    def fire(c):
        buf = gbufs[c % _NBUF]
        sem = gsems[c % _NBUF]
        tvec = lax.shift_right_logical(xv[pl.ds(c * _CHUNK, _CHUNK)], 3)
        for jj in range(_CHUNK):
            pltpu.async_copy(t3.at[tvec[jj]], buf.at[jj], sem)

    def drain(c):
        pltpu.make_async_copy(
            t3.at[pl.ds(0, _CHUNK)], gbufs[c % _NBUF], gsems[c % _NBUF]
        ).wait()

    fire(0)
    fire(1)
    for c in range(_NCHUNKS):
        drain(c)
        if c + 2 < _NCHUNKS:
            fire(c + 2)
        if c == 0:
            pos_cp.wait()
        gath = gbufs[c % _NBUF]
        svec = lax.bitwise_and(xv[pl.ds(c * _CHUNK, _CHUNK)], 7)
        for jo in range(_CHUNK):
            j = c * _CHUNK + jo
            s = svec[jo]
            for cc in range(_DIM // _LANES):
                sl = pl.ds(cc * _LANES, _LANES)
                rows_v[j, sl] = gath[jo, s, sl] + posb[j, sl]

    pltpu.sync_copy(rows_v, out_hbm.at[pl.ds(base, _BPW), :])


def kernel(x, table, pos_enc):
    table3 = table.reshape(_VOCAB // 8, 8, _DIM)
    mesh = plsc.VectorSubcoreMesh(core_axis_name="c", subcore_axis_name="s")
    f = pl.kernel(
        _emb_body,
        mesh=mesh,
        out_type=jax.ShapeDtypeStruct((_CONTEXT, _DIM), jnp.float32),
        scratch_types=[
            pltpu.VMEM((_BPW,), jnp.int32),
            pltpu.VMEM((_CHUNK, 8, _DIM), jnp.float32),
            pltpu.VMEM((_CHUNK, 8, _DIM), jnp.float32),
            pltpu.VMEM((_CHUNK, 8, _DIM), jnp.float32),
            pltpu.VMEM((_BPW, _DIM), jnp.float32),
            pltpu.VMEM((_BPW, _DIM), jnp.float32),
            pltpu.SemaphoreType.DMA,
            pltpu.SemaphoreType.DMA,
            pltpu.SemaphoreType.DMA,
            pltpu.SemaphoreType.DMA,
            pltpu.SemaphoreType.DMA,
        ],
    )
    return f(x, table3, pos_enc)


# speedup vs baseline: 2.3724x; 1.0006x over previous
import jax
import jax.numpy as jnp
from jax import lax
from jax.experimental import pallas as pl
from jax.experimental.pallas import tpu as pltpu
from jax.experimental.pallas import tpu_sc as plsc

_CONTEXT = 8192
_DIM = 64
_VOCAB = 1000000
_LANES = 16
_NUM_WORKERS = 32
_BPW = _CONTEXT // _NUM_WORKERS
_CHUNK = 16
_NCHUNKS = _BPW // _CHUNK


_NBUF = 3


def _emb_body(x_hbm, t_hbm, pos_hbm, out_hbm,
              xv, tidv, gb0, gb1, gb2, rows_v, posb, gsem0, gsem1, gsem2,
              psem, xsem):
    wid = lax.axis_index("s") * 2 + lax.axis_index("c")
    base = wid * _BPW

    t3 = t_hbm

    pltpu.async_copy(x_hbm.at[pl.ds(base, _BPW)], xv, xsem).wait()
    pos_cp = pltpu.async_copy(pos_hbm.at[pl.ds(base, _BPW), :], posb, psem)

    gbufs = (gb0, gb1, gb2)
    gsems = (gsem0, gsem1, gsem2)

    def tid_step(i, carry):
        sl = pl.ds(i * _LANES, _LANES)
        tidv[sl] = lax.shift_right_logical(xv[sl], 3)
        return carry

    lax.fori_loop(0, _BPW // _LANES, tid_step, 0)

    def fire(c):
        buf = gbufs[c % _NBUF]
        sem = gsems[c % _NBUF]
        tvec = tidv[pl.ds(c * _CHUNK, _CHUNK)]
        for jj in range(_CHUNK):
            pltpu.async_copy(t3.at[tvec[jj]], buf.at[jj], sem)

    def drain(c):
        pltpu.make_async_copy(
            t3.at[pl.ds(0, _CHUNK)], gbufs[c % _NBUF], gsems[c % _NBUF]
        ).wait()

    fire(0)
    fire(1)
    for c in range(_NCHUNKS):
        drain(c)
        if c + 2 < _NCHUNKS:
            fire(c + 2)
        if c == 0:
            pos_cp.wait()
        gath = gbufs[c % _NBUF]
        svec = lax.bitwise_and(xv[pl.ds(c * _CHUNK, _CHUNK)], 7)
        for jo in range(_CHUNK):
            j = c * _CHUNK + jo
            s = svec[jo]
            for cc in range(_DIM // _LANES):
                sl = pl.ds(cc * _LANES, _LANES)
                rows_v[j, sl] = gath[jo, s, sl] + posb[j, sl]

    pltpu.sync_copy(rows_v, out_hbm.at[pl.ds(base, _BPW), :])


def kernel(x, table, pos_enc):
    table3 = table.reshape(_VOCAB // 8, 8, _DIM)
    mesh = plsc.VectorSubcoreMesh(core_axis_name="c", subcore_axis_name="s")
    f = pl.kernel(
        _emb_body,
        mesh=mesh,
        out_type=jax.ShapeDtypeStruct((_CONTEXT, _DIM), jnp.float32),
        scratch_types=[
            pltpu.VMEM((_BPW,), jnp.int32),
            pltpu.VMEM((_BPW,), jnp.int32),
            pltpu.VMEM((_CHUNK, 8, _DIM), jnp.float32),
            pltpu.VMEM((_CHUNK, 8, _DIM), jnp.float32),
            pltpu.VMEM((_CHUNK, 8, _DIM), jnp.float32),
            pltpu.VMEM((_BPW, _DIM), jnp.float32),
            pltpu.VMEM((_BPW, _DIM), jnp.float32),
            pltpu.SemaphoreType.DMA,
            pltpu.SemaphoreType.DMA,
            pltpu.SemaphoreType.DMA,
            pltpu.SemaphoreType.DMA,
            pltpu.SemaphoreType.DMA,
        ],
    )
    return f(x, table3, pos_enc)
